# baseline (speedup 1.0000x reference)
import functools

import jax
import jax.numpy as jnp
from jax import lax
from jax.experimental import pallas as pl
from jax.experimental.pallas import tpu as pltpu
from jax.experimental.pallas import tpu_sc as plsc

NC = 2
NS = 16
L = 16
NW = NC * NS
B = 16384
D = 64
PR = 128
CHUNK = 128
CPW = B // NW // CHUNK
NROW = B // CHUNK
BPW = B // NW

_mesh = plsc.VectorSubcoreMesh(
    core_axis_name="c", subcore_axis_name="s", num_cores=NC, num_subcores=NS
)


@functools.partial(
    pl.kernel,
    out_type=jax.ShapeDtypeStruct((NW * L,), jnp.float32),
    mesh=_mesh,
    scratch_types=[
        pltpu.VMEM((CPW, CHUNK), jnp.int32),
        pltpu.VMEM((CPW, CHUNK), jnp.int32),
        pltpu.VMEM((BPW,), jnp.int32),
        pltpu.VMEM((BPW,), jnp.int32),
        pltpu.VMEM((BPW,), jnp.int32),
        pltpu.VMEM((BPW,), jnp.int32),
        pltpu.VMEM((2, CHUNK, PR), jnp.float32),
        pltpu.VMEM((2, CHUNK, PR), jnp.float32),
        pltpu.VMEM((L,), jnp.float32),
        pltpu.SemaphoreType.DMA,
    ],
    compiler_params=pltpu.CompilerParams(needs_layout_passes=False),
)
def _pair_gather_reduce(idxu_hbm, idxb_hbm, u2_hbm, b2_hbm, part_hbm,
                        pu_v, pb_v, hu_v, hb_v, idxu_v, idxb_v,
                        urows, brows, accv, sem):
    wid = lax.axis_index("s") * NC + lax.axis_index("c")
    pltpu.sync_copy(idxu_hbm.at[pl.ds(wid * BPW, BPW)], idxu_v)
    pltpu.sync_copy(idxb_hbm.at[pl.ds(wid * BPW, BPW)], idxb_v)
    for t in range(BPW // L):
        sl = pl.ds(t * L, L)
        j, i = t // (CHUNK // L), t % (CHUNK // L)
        pu_v[j, pl.ds(i * L, L)] = jax.lax.shift_right_logical(idxu_v[sl], 1)
        pb_v[j, pl.ds(i * L, L)] = jax.lax.shift_right_logical(idxb_v[sl], 1)
        hu_v[sl] = (idxu_v[sl] & 1) * D
        hb_v[sl] = (idxb_v[sl] & 1) * D

    def fire(j):
        pltpu.async_copy(u2_hbm.at[pu_v.at[j]], urows.at[j % 2], sem)
        pltpu.async_copy(b2_hbm.at[pb_v.at[j]], brows.at[j % 2], sem)

    def drain():
        pltpu.make_async_copy(u2_hbm.at[pl.ds(0, CHUNK)], urows.at[0], sem).wait()
        pltpu.make_async_copy(b2_hbm.at[pl.ds(0, CHUNK)], brows.at[0], sem).wait()

    fire(0)
    acc = jnp.zeros((L,), jnp.float32)
    for j in range(CPW):
        drain()
        if j + 1 < CPW:
            fire(j + 1)
        buf = j % 2

        def row_body(i, a, j=j, buf=buf):
            ridx = jax.lax.broadcast(j * CHUNK + i, (L,))
            hu16 = plsc.load_gather(hu_v, [ridx])
            hb16 = plsc.load_gather(hb_v, [ridx])
            bufv = jax.lax.broadcast(buf, (L,))
            iv = jax.lax.broadcast(i, (L,))
            lane = lax.iota(jnp.int32, L)
            for c in range(D // L):
                usel = plsc.load_gather(urows, [bufv, iv, hu16 + (c * L) + lane])
                bsel = plsc.load_gather(brows, [bufv, iv, hb16 + (c * L) + lane])
                a = a + usel * bsel
            return a

        acc = lax.fori_loop(0, CHUNK, row_body, acc)
    accv[...] = acc
    pltpu.sync_copy(accv, part_hbm.at[pl.ds(wid * L, L)])


@functools.partial(
    pl.kernel,
    out_type=(
        jax.ShapeDtypeStruct((NROW, CHUNK), jnp.float32),
        jax.ShapeDtypeStruct((NROW, CHUNK), jnp.float32),
    ),
    mesh=_mesh,
    scratch_types=[
        pltpu.VMEM((CPW, CHUNK), jnp.int32),
        pltpu.VMEM((CPW, CHUNK), jnp.int32),
        pltpu.VMEM((CPW, CHUNK), jnp.float32),
        pltpu.VMEM((CPW, CHUNK), jnp.float32),
        pltpu.SemaphoreType.DMA,
    ],
    compiler_params=pltpu.CompilerParams(use_tc_tiling_on_sc=False),
)
def _bias_gather(idxu_hbm, idxb_hbm, ubias_hbm, bbias_hbm,
                 ubg_hbm, bbg_hbm,
                 idxu_v, idxb_v, ubv, bbv, sem):
    wid = lax.axis_index("s") * NC + lax.axis_index("c")
    base = wid * CPW
    pltpu.sync_copy(idxu_hbm.at[pl.ds(base, CPW)], idxu_v)
    pltpu.sync_copy(idxb_hbm.at[pl.ds(base, CPW)], idxb_v)
    copies = []
    for j in range(CPW):
        copies.append(pltpu.async_copy(ubias_hbm.at[idxu_v.at[j]], ubv.at[j], sem))
        copies.append(pltpu.async_copy(bbias_hbm.at[idxb_v.at[j]], bbv.at[j], sem))
    for c in copies:
        c.wait()
    pltpu.sync_copy(ubv, ubg_hbm.at[pl.ds(base, CPW)])
    pltpu.sync_copy(bbv, bbg_hbm.at[pl.ds(base, CPW)])


def _finish_body(p_ref, ub_ref, bb_ref, o_ref):
    s = jnp.sum(p_ref[...])
    x = s + ub_ref[...] + bb_ref[...]
    o_ref[...] = 1.0 / (1.0 + jnp.exp(-x))


def kernel(inputs, user_emb_table, user_bias_table, blog_emb_table, blog_bias_table):
    idx = inputs.astype(jnp.int32)
    idxu = idx[:, 0]
    idxb = idx[:, 1]
    part = _pair_gather_reduce(
        idxu, idxb,
        user_emb_table.reshape(500000, PR),
        blog_emb_table.reshape(500000, PR),
    )
    ubg, bbg = _bias_gather(
        idxu.reshape(NROW, CHUNK), idxb.reshape(NROW, CHUNK),
        user_bias_table.reshape(-1), blog_bias_table.reshape(-1),
    )
    out = pl.pallas_call(
        _finish_body,
        out_shape=jax.ShapeDtypeStruct((NROW, CHUNK), jnp.float32),
    )(part.reshape(NW * L // CHUNK, CHUNK), ubg, bbg)
    return out.reshape(B, 1)
